# Initial kernel scaffold; baseline (speedup 1.0000x reference)
import jax
import jax.numpy as jnp
from jax.experimental import pallas as pl


def kernel(x, gemm_edges, W, b):
    raise NotImplementedError("write your pallas kernel here")


# speedup vs baseline: 6.7913x; 6.7913x over previous
import functools

import jax
import jax.numpy as jnp
from jax import lax
from jax.experimental import pallas as pl
from jax.experimental.pallas import tpu as pltpu
from jax.experimental.pallas import tpu_sc as plsc

_NC, _NS = 2, 16
_NW = _NC * _NS


def _sc_gather(xT, i1, i2, i3, i4):
    E, C = xT.shape
    per_w = E // _NW
    CH = 128
    n_full = per_w // CH
    tail = per_w - n_full * CH

    mesh = plsc.VectorSubcoreMesh(core_axis_name="c", subcore_axis_name="s")

    @functools.partial(
        pl.kernel,
        mesh=mesh,
        out_type=jax.ShapeDtypeStruct((4, E, C), jnp.float32),
        scratch_types=[
            pltpu.VMEM((per_w,), jnp.int32),
            pltpu.VMEM((per_w,), jnp.int32),
            pltpu.VMEM((per_w,), jnp.int32),
            pltpu.VMEM((per_w,), jnp.int32),
            pltpu.VMEM((4, CH, C), jnp.float32),
            pltpu.SemaphoreType.DMA,
        ],
    )
    def k(xT_hbm, i1_hbm, i2_hbm, i3_hbm, i4_hbm, out_hbm,
          iv1, iv2, iv3, iv4, rows_v, sem):
        wid = lax.axis_index("s") * _NC + lax.axis_index("c")
        base = pl.multiple_of(wid * per_w, 8)
        idx_vs = (iv1, iv2, iv3, iv4)
        for j4, ik in enumerate((i1_hbm, i2_hbm, i3_hbm, i4_hbm)):
            pltpu.sync_copy(ik.at[pl.ds(base, per_w)], idx_vs[j4])

        def chunk(j, carry):
            off = pl.multiple_of(j * CH, CH)
            cps = [
                pltpu.async_copy(
                    xT_hbm.at[idx_vs[j4].at[pl.ds(off, CH)]], rows_v.at[j4], sem
                )
                for j4 in range(4)
            ]
            for cp in cps:
                cp.wait()
            for j4 in range(4):
                pltpu.sync_copy(
                    rows_v.at[j4], out_hbm.at[j4, pl.ds(base + off, CH)]
                )
            return carry

        lax.fori_loop(0, n_full, chunk, 0)

        if tail:
            off = n_full * CH
            cps = [
                pltpu.async_copy(
                    xT_hbm.at[idx_vs[j4].at[pl.ds(off, tail)]],
                    rows_v.at[j4, pl.ds(0, tail)],
                    sem,
                )
                for j4 in range(4)
            ]
            for cp in cps:
                cp.wait()
            for j4 in range(4):
                pltpu.sync_copy(
                    rows_v.at[j4, pl.ds(0, tail)],
                    out_hbm.at[j4, pl.ds(base + off, tail)],
                )

    return k(xT, i1, i2, i3, i4)


def _tc_conv(xT, f4, Wt, b2):
    E, C = xT.shape
    O = Wt.shape[2]
    BLK = 640

    def body(x_ref, f_ref, w_ref, b_ref, o_ref):
        xb = x_ref[...]
        f1 = f_ref[0]
        f2 = f_ref[1]
        f3 = f_ref[2]
        f4_ = f_ref[3]
        s13 = f1 + f3
        s24 = f2 + f4_
        d13 = jnp.abs(f1 - f3)
        d24 = jnp.abs(f2 - f4_)
        w = w_ref[...]

        def mm(v, kk):
            return lax.dot_general(
                v, w[kk], (((1,), (0,)), ((), ())),
                preferred_element_type=jnp.float32,
            )

        acc = mm(xb, 0) + mm(s13, 1) + mm(s24, 2) + mm(d13, 3) + mm(d24, 4)
        o_ref[...] = acc + b_ref[...]

    return pl.pallas_call(
        body,
        grid=(E // BLK,),
        in_specs=[
            pl.BlockSpec((BLK, C), lambda i: (i, 0)),
            pl.BlockSpec((4, BLK, C), lambda i: (0, i, 0)),
            pl.BlockSpec((5, C, O), lambda i: (0, 0, 0)),
            pl.BlockSpec((1, O), lambda i: (0, 0)),
        ],
        out_specs=pl.BlockSpec((BLK, O), lambda i: (i, 0)),
        out_shape=jax.ShapeDtypeStruct((E, O), jnp.float32),
    )(xT, f4, Wt, b2)


def kernel(x, gemm_edges, W, b):
    xT = jnp.transpose(x[0, :, :, 0])
    ge = gemm_edges[0].astype(jnp.int32)
    f4 = _sc_gather(xT, ge[:, 0], ge[:, 1], ge[:, 2], ge[:, 3])
    Wt = jnp.transpose(W[:, :, 0, :], (2, 1, 0))
    out = _tc_conv(xT, f4, Wt, b[None, :])
    return jnp.transpose(out)[None, :, :, None]
